# Initial kernel scaffold; baseline (speedup 1.0000x reference)
import jax
import jax.numpy as jnp
from jax.experimental import pallas as pl


def kernel(inputs, emb, W1, b1, W2, b2):
    raise NotImplementedError("write your pallas kernel here")


# speedup vs baseline: 1.3572x; 1.3572x over previous
import functools

import jax
import jax.numpy as jnp
from jax import lax
from jax.experimental import pallas as pl
from jax.experimental.pallas import tpu as pltpu
from jax.experimental.pallas import tpu_sc as plsc

VOCAB = 100000
EMBED_DIM = 32
HIDDEN = 128
BATCH = 1024
CTX = 200

_L = 16


_C0 = 128
_C1 = CTX - _C0


def _sc_pool_kernel(
    emb_hbm, idx_hbm, out_hbm, idx0_v, idx1_v, rows0_v, rows1_v, pooled_v, sem
):
    nc = 2
    b_per_w = BATCH // 32
    wid = lax.axis_index("s") * nc + lax.axis_index("c")
    base = wid * b_per_w

    inv = jnp.full((_L,), 1.0 / CTX, dtype=jnp.float32)

    def row_body(i, _):
        row = base + i
        pltpu.sync_copy(idx_hbm.at[row, pl.ds(0, _C0)], idx0_v)
        pltpu.sync_copy(idx_hbm.at[row, pl.ds(_C0, _C1)], idx1_v)
        pltpu.async_copy(emb_hbm.at[idx0_v], rows0_v, sem).wait()
        pltpu.async_copy(emb_hbm.at[idx1_v], rows1_v, sem).wait()

        def acc0_body(j, carry):
            a0, a1 = carry
            for u in range(8):
                r = j * 8 + u
                a0 = a0 + rows0_v[r, 0:_L]
                a1 = a1 + rows0_v[r, _L : 2 * _L]
            return (a0, a1)

        def acc1_body(j, carry):
            a0, a1 = carry
            for u in range(8):
                r = j * 8 + u
                a0 = a0 + rows1_v[r, 0:_L]
                a1 = a1 + rows1_v[r, _L : 2 * _L]
            return (a0, a1)

        z = jnp.zeros((_L,), dtype=jnp.float32)
        a0, a1 = lax.fori_loop(0, _C0 // 8, acc0_body, (z, z))
        a0, a1 = lax.fori_loop(0, _C1 // 8, acc1_body, (a0, a1))
        pooled_v[i, 0:_L] = a0 * inv
        pooled_v[i, _L : 2 * _L] = a1 * inv
        return 0

    lax.fori_loop(0, b_per_w, row_body, 0)
    pltpu.sync_copy(pooled_v, out_hbm.at[pl.ds(base, b_per_w)])


def _sc_pool(emb, idx):
    b_per_w = BATCH // 32
    mesh = plsc.VectorSubcoreMesh(core_axis_name="c", subcore_axis_name="s")
    return pl.kernel(
        _sc_pool_kernel,
        mesh=mesh,
        out_type=jax.ShapeDtypeStruct((BATCH, EMBED_DIM), jnp.float32),
        scratch_types=[
            pltpu.VMEM((_C0,), jnp.int32),
            pltpu.VMEM((_C1,), jnp.int32),
            pltpu.VMEM((_C0, EMBED_DIM), jnp.float32),
            pltpu.VMEM((_C1, EMBED_DIM), jnp.float32),
            pltpu.VMEM((b_per_w, EMBED_DIM), jnp.float32),
            pltpu.SemaphoreType.DMA,
        ],
        compiler_params=pltpu.CompilerParams(use_tc_tiling_on_sc=False),
    )(emb, idx)


def _mlp_block(pooled_ref, w1_ref, b1_ref, w2_ref, b2_ref, out_ref, h_ref):
    @pl.when(pl.program_id(0) == 0)
    def _():
        h = (
            jnp.dot(pooled_ref[...], w1_ref[...], preferred_element_type=jnp.float32)
            + b1_ref[...]
        )
        h_ref[...] = jnp.maximum(h, 0.0)

    out_ref[...] = (
        jnp.dot(h_ref[...], w2_ref[...], preferred_element_type=jnp.float32)
        + b2_ref[...]
    )


def _tc_mlp(pooled, W1, b1, W2, b2):
    tw = 2048
    grid = (pl.cdiv(VOCAB, tw),)
    return pl.pallas_call(
        _mlp_block,
        grid=grid,
        in_specs=[
            pl.BlockSpec((BATCH, EMBED_DIM), lambda j: (0, 0)),
            pl.BlockSpec((EMBED_DIM, HIDDEN), lambda j: (0, 0)),
            pl.BlockSpec((1, HIDDEN), lambda j: (0, 0)),
            pl.BlockSpec((HIDDEN, tw), lambda j: (0, j)),
            pl.BlockSpec((1, tw), lambda j: (0, j)),
        ],
        out_specs=pl.BlockSpec((BATCH, tw), lambda j: (0, j)),
        out_shape=jax.ShapeDtypeStruct((BATCH, VOCAB), jnp.float32),
        scratch_shapes=[pltpu.VMEM((BATCH, HIDDEN), jnp.float32)],
        compiler_params=pltpu.CompilerParams(
            dimension_semantics=("arbitrary",),
        ),
    )(pooled, W1, b1.reshape(1, HIDDEN), W2, b2.reshape(1, VOCAB))


def kernel(inputs, emb, W1, b1, W2, b2):
    pooled = _sc_pool(emb, inputs.astype(jnp.int32))
    return _tc_mlp(pooled, W1, b1, W2, b2)


# speedup vs baseline: 1.3600x; 1.0021x over previous
import functools

import jax
import jax.numpy as jnp
from jax import lax
from jax.experimental import pallas as pl
from jax.experimental.pallas import tpu as pltpu
from jax.experimental.pallas import tpu_sc as plsc

VOCAB = 100000
EMBED_DIM = 32
HIDDEN = 128
BATCH = 1024
CTX = 200

_L = 16


_C0 = 128
_C1 = CTX - _C0


def _sc_pool_kernel(
    emb_hbm, idx_hbm, out_hbm, idx0_v, idx1_v, rows0_v, rows1_v, pooled_v, sem
):
    nc = 2
    b_per_w = BATCH // 32
    wid = lax.axis_index("s") * nc + lax.axis_index("c")
    base = wid * b_per_w

    inv = jnp.full((_L,), 1.0 / CTX, dtype=jnp.float32)

    def row_body(i, _):
        row = base + i
        pltpu.sync_copy(idx_hbm.at[row, pl.ds(0, _C0)], idx0_v)
        pltpu.sync_copy(idx_hbm.at[row, pl.ds(_C0, _C1)], idx1_v)
        pltpu.async_copy(emb_hbm.at[idx0_v], rows0_v, sem).wait()
        pltpu.async_copy(emb_hbm.at[idx1_v], rows1_v, sem).wait()

        def acc0_body(j, carry):
            a0, a1 = carry
            for u in range(8):
                r = j * 8 + u
                a0 = a0 + rows0_v[r, 0:_L]
                a1 = a1 + rows0_v[r, _L : 2 * _L]
            return (a0, a1)

        def acc1_body(j, carry):
            a0, a1 = carry
            for u in range(8):
                r = j * 8 + u
                a0 = a0 + rows1_v[r, 0:_L]
                a1 = a1 + rows1_v[r, _L : 2 * _L]
            return (a0, a1)

        z = jnp.zeros((_L,), dtype=jnp.float32)
        a0, a1 = lax.fori_loop(0, _C0 // 8, acc0_body, (z, z))
        a0, a1 = lax.fori_loop(0, _C1 // 8, acc1_body, (a0, a1))
        pooled_v[i, 0:_L] = a0 * inv
        pooled_v[i, _L : 2 * _L] = a1 * inv
        return 0

    lax.fori_loop(0, b_per_w, row_body, 0)
    pltpu.sync_copy(pooled_v, out_hbm.at[pl.ds(base, b_per_w)])


def _sc_pool(emb, idx):
    b_per_w = BATCH // 32
    mesh = plsc.VectorSubcoreMesh(core_axis_name="c", subcore_axis_name="s")
    return pl.kernel(
        _sc_pool_kernel,
        mesh=mesh,
        out_type=jax.ShapeDtypeStruct((BATCH, EMBED_DIM), jnp.float32),
        scratch_types=[
            pltpu.VMEM((_C0,), jnp.int32),
            pltpu.VMEM((_C1,), jnp.int32),
            pltpu.VMEM((_C0, EMBED_DIM), jnp.float32),
            pltpu.VMEM((_C1, EMBED_DIM), jnp.float32),
            pltpu.VMEM((b_per_w, EMBED_DIM), jnp.float32),
            pltpu.SemaphoreType.DMA,
        ],
        compiler_params=pltpu.CompilerParams(use_tc_tiling_on_sc=False),
    )(emb, idx)


def _mlp_block(pooled_ref, w1_ref, b1_ref, w2_ref, b2_ref, out_ref, h_ref):
    @pl.when(pl.program_id(0) == 0)
    def _():
        h = (
            jnp.dot(pooled_ref[...], w1_ref[...], preferred_element_type=jnp.float32)
            + b1_ref[...]
        )
        h_ref[...] = jnp.maximum(h, 0.0)

    out_ref[...] = (
        jnp.dot(h_ref[...], w2_ref[...], preferred_element_type=jnp.float32)
        + b2_ref[...]
    )


def _tc_mlp(pooled, W1, b1, W2, b2):
    tw = 4096
    grid = (pl.cdiv(VOCAB, tw),)
    return pl.pallas_call(
        _mlp_block,
        grid=grid,
        in_specs=[
            pl.BlockSpec((BATCH, EMBED_DIM), lambda j: (0, 0)),
            pl.BlockSpec((EMBED_DIM, HIDDEN), lambda j: (0, 0)),
            pl.BlockSpec((1, HIDDEN), lambda j: (0, 0)),
            pl.BlockSpec((HIDDEN, tw), lambda j: (0, j)),
            pl.BlockSpec((1, tw), lambda j: (0, j)),
        ],
        out_specs=pl.BlockSpec((BATCH, tw), lambda j: (0, j)),
        out_shape=jax.ShapeDtypeStruct((BATCH, VOCAB), jnp.float32),
        scratch_shapes=[pltpu.VMEM((BATCH, HIDDEN), jnp.float32)],
        compiler_params=pltpu.CompilerParams(
            dimension_semantics=("arbitrary",),
        ),
    )(pooled, W1, b1.reshape(1, HIDDEN), W2, b2.reshape(1, VOCAB))


def kernel(inputs, emb, W1, b1, W2, b2):
    pooled = _sc_pool(emb, inputs.astype(jnp.int32))
    return _tc_mlp(pooled, W1, b1, W2, b2)
